# baseline (speedup 1.0000x reference)
import functools
import jax
import jax.numpy as jnp
from jax import lax
from jax.experimental import pallas as pl
from jax.experimental.pallas import tpu as pltpu
from jax.experimental.pallas import tpu_sc as plsc

BATCH = 16384
SEQ = 50
SEQ_PAD = 56
EMBED_DIM = 64
LANE_PAD = 128
NUM_INDICES = BATCH * SEQ
NUM_WORKERS = 32
BPC = 16
CHUNK = BPC * SEQ
BATCH_PER_WORKER = BATCH // NUM_WORKERS
NUM_CHUNKS = BATCH_PER_WORKER // BPC
VLEN = 16


def _sc_gather(table, idx):
    mesh = plsc.VectorSubcoreMesh(core_axis_name="c", subcore_axis_name="s")

    @functools.partial(
        pl.kernel,
        mesh=mesh,
        out_type=jax.ShapeDtypeStruct((BATCH, SEQ_PAD, LANE_PAD), table.dtype),
        scratch_types=[
            pltpu.VMEM((2, BPC, SEQ), jnp.int32),
            pltpu.VMEM((2, CHUNK), jnp.int32),
            pltpu.VMEM((2, CHUNK, EMBED_DIM), jnp.float32),
            pltpu.SemaphoreType.DMA,
            pltpu.SemaphoreType.DMA,
            pltpu.SemaphoreType.DMA,
        ],
        compiler_params=pltpu.CompilerParams(
            use_tc_tiling_on_sc=False, needs_layout_passes=False
        ),
    )
    def sc_gather(
        table_hbm, idx_hbm, out_hbm, slab_v, idx_v, rows_v, sem_s, sem_g, sem_w
    ):
        wid = lax.axis_index("s") * 2 + lax.axis_index("c")
        base_batch = wid * BATCH_PER_WORKER

        def slab_src(c):
            return idx_hbm.at[pl.ds(base_batch + c * BPC, BPC), pl.ds(0, SEQ)]

        lanes = lax.iota(jnp.int32, VLEN)

        def compact(buf):
            for i in range(CHUNK // VLEN):
                k = lanes + (i * VLEN)
                rows = k // SEQ
                cols = k - rows * SEQ
                idx_v[buf, pl.ds(i * VLEN, VLEN)] = plsc.load_gather(
                    slab_v.at[buf], [rows, cols]
                )

        def writeback_descs(buf, c):
            descs = []
            for b in range(BPC):
                bid = base_batch + c * BPC + b
                descs.append(
                    pltpu.make_async_copy(
                        rows_v.at[buf, pl.ds(b * SEQ, SEQ)],
                        out_hbm.at[bid, pl.ds(0, SEQ), pl.ds(0, EMBED_DIM)],
                        sem_w,
                    )
                )
            return descs

        pltpu.sync_copy(slab_src(0), slab_v.at[0])
        compact(0)
        pltpu.async_copy(slab_src(1), slab_v.at[1], sem_s)

        @pl.loop(0, NUM_CHUNKS // 2)
        def _(g):
            for parity in (0, 1):
                c = g * 2 + parity
                buf = parity
                nxt = 1 - parity

                @pl.when(g >= 1)
                def _():
                    for d in writeback_descs(buf, c - 2):
                        d.wait()

                gather = pltpu.async_copy(
                    table_hbm.at[idx_v.at[buf]], rows_v.at[buf], sem_g
                )

                def prep_next():
                    pltpu.make_async_copy(
                        slab_src(0), slab_v.at[nxt], sem_s
                    ).wait()
                    compact(nxt)

                def prefetch(c2):
                    pltpu.async_copy(slab_src(c2), slab_v.at[buf], sem_s)

                if parity == 0:
                    prep_next()

                    @pl.when(g < NUM_CHUNKS // 2 - 1)
                    def _():
                        prefetch(c + 2)
                else:
                    @pl.when(g < NUM_CHUNKS // 2 - 1)
                    def _():
                        prep_next()

                    @pl.when(g < NUM_CHUNKS // 2 - 1)
                    def _():
                        prefetch(c + 2)

                gather.wait()

                for d in writeback_descs(buf, c):
                    d.start()

        for c in (NUM_CHUNKS - 2, NUM_CHUNKS - 1):
            for d in writeback_descs(c % 2, c):
                d.wait()

    return sc_gather(table, idx)


def kernel(x, table):
    idx = x

    @jax.jit
    def run(table, idx):
        mid = _sc_gather(table, idx)
        return mid[:, :SEQ, :EMBED_DIM]

    return run(table, idx)


# speedup vs baseline: 1.0237x; 1.0237x over previous
import functools
import jax
import jax.numpy as jnp
from jax import lax
from jax.experimental import pallas as pl
from jax.experimental.pallas import tpu as pltpu
from jax.experimental.pallas import tpu_sc as plsc

BATCH = 16384
SEQ = 50
SEQ_PAD = 56
EMBED_DIM = 64
LANE_PAD = 128
NUM_INDICES = BATCH * SEQ
NUM_WORKERS = 32
BPC = 16
CHUNK = BPC * SEQ
BATCH_PER_WORKER = BATCH // NUM_WORKERS
NUM_CHUNKS = BATCH_PER_WORKER // BPC
TC_BLOCK_B = 64


def _sc_gather(table, idx):
    mesh = plsc.VectorSubcoreMesh(core_axis_name="c", subcore_axis_name="s")

    @functools.partial(
        pl.kernel,
        mesh=mesh,
        out_type=jax.ShapeDtypeStruct((BATCH, SEQ_PAD, LANE_PAD), table.dtype),
        scratch_types=[
            pltpu.VMEM((2, CHUNK), jnp.int32),
            pltpu.VMEM((2, CHUNK, EMBED_DIM), jnp.float32),
            pltpu.SemaphoreType.DMA,
            pltpu.SemaphoreType.DMA,
            pltpu.SemaphoreType.DMA,
        ],
        compiler_params=pltpu.CompilerParams(use_tc_tiling_on_sc=False),
    )
    def sc_gather(table_hbm, idx_hbm, out_hbm, idx_v, rows_v, sem_i, sem_g, sem_w):
        wid = lax.axis_index("s") * 2 + lax.axis_index("c")
        base_row = wid * BATCH_PER_WORKER * SEQ
        base_batch = wid * BATCH_PER_WORKER

        def writeback_descs(buf, c):
            descs = []
            for b in range(BPC):
                bid = base_batch + c * BPC + b
                descs.append(
                    pltpu.make_async_copy(
                        rows_v.at[buf, pl.ds(b * SEQ, SEQ)],
                        out_hbm.at[bid, pl.ds(0, SEQ), pl.ds(0, EMBED_DIM)],
                        sem_w,
                    )
                )
            return descs

        pltpu.sync_copy(idx_hbm.at[pl.ds(base_row, CHUNK)], idx_v.at[0])

        @pl.loop(0, NUM_CHUNKS // 2)
        def _(g):
            for parity in (0, 1):
                c = g * 2 + parity
                buf = parity
                nxt = 1 - parity
                if parity == 0:
                    pltpu.async_copy(
                        idx_hbm.at[pl.ds(base_row + (c + 1) * CHUNK, CHUNK)],
                        idx_v.at[nxt],
                        sem_i,
                    )
                else:
                    @pl.when(g < NUM_CHUNKS // 2 - 1)
                    def _():
                        pltpu.async_copy(
                            idx_hbm.at[pl.ds(base_row + (c + 1) * CHUNK, CHUNK)],
                            idx_v.at[nxt],
                            sem_i,
                        )

                @pl.when(g >= 1)
                def _():
                    for d in writeback_descs(buf, c - 2):
                        d.wait()

                pltpu.async_copy(
                    table_hbm.at[idx_v.at[buf]], rows_v.at[buf], sem_g
                ).wait()

                for d in writeback_descs(buf, c):
                    d.start()

                if parity == 0:
                    pltpu.make_async_copy(
                        idx_hbm.at[pl.ds(0, CHUNK)], idx_v.at[nxt], sem_i
                    ).wait()
                else:
                    @pl.when(g < NUM_CHUNKS // 2 - 1)
                    def _():
                        pltpu.make_async_copy(
                            idx_hbm.at[pl.ds(0, CHUNK)], idx_v.at[nxt], sem_i
                        ).wait()

        for c in (NUM_CHUNKS - 2, NUM_CHUNKS - 1):
            for d in writeback_descs(c % 2, c):
                d.wait()

    return sc_gather(table, idx)


def _tc_repack(mid):
    def body(i_ref, o_ref):
        o_ref[...] = i_ref[:, :SEQ, :EMBED_DIM]

    return pl.pallas_call(
        body,
        out_shape=jax.ShapeDtypeStruct((BATCH, SEQ, EMBED_DIM), mid.dtype),
        grid=(BATCH // TC_BLOCK_B,),
        in_specs=[
            pl.BlockSpec((TC_BLOCK_B, SEQ_PAD, LANE_PAD), lambda i: (i, 0, 0))
        ],
        out_specs=pl.BlockSpec(
            (TC_BLOCK_B, SEQ, EMBED_DIM), lambda i: (i, 0, 0)
        ),
    )(mid)


def kernel(x, table):
    idx = x.reshape(NUM_INDICES)

    @jax.jit
    def run(table, idx):
        mid = _sc_gather(table, idx)
        return mid[:, :SEQ, :EMBED_DIM]

    return run(table, idx)


# speedup vs baseline: 1.0257x; 1.0020x over previous
import functools
import jax
import jax.numpy as jnp
from jax import lax
from jax.experimental import pallas as pl
from jax.experimental.pallas import tpu as pltpu
from jax.experimental.pallas import tpu_sc as plsc

BATCH = 16384
SEQ = 50
SEQ_PAD = 56
EMBED_DIM = 64
LANE_PAD = 128
NUM_INDICES = BATCH * SEQ
NUM_WORKERS = 32
BPC = 16
CHUNK = BPC * SEQ
BATCH_PER_WORKER = BATCH // NUM_WORKERS
NUM_CHUNKS = BATCH_PER_WORKER // BPC


def _sc_gather(table, idx):
    mesh = plsc.VectorSubcoreMesh(core_axis_name="c", subcore_axis_name="s")

    @functools.partial(
        pl.kernel,
        mesh=mesh,
        out_type=jax.ShapeDtypeStruct((BATCH, SEQ_PAD, LANE_PAD), table.dtype),
        scratch_types=[
            pltpu.VMEM((2, CHUNK), jnp.int32),
            pltpu.VMEM((2, CHUNK, EMBED_DIM), jnp.float32),
            pltpu.SemaphoreType.DMA,
            pltpu.SemaphoreType.DMA,
            pltpu.SemaphoreType.DMA,
        ],
        compiler_params=pltpu.CompilerParams(use_tc_tiling_on_sc=False),
    )
    def sc_gather(table_hbm, idx_hbm, out_hbm, idx_v, rows_v, sem_i, sem_g, sem_w):
        wid = lax.axis_index("s") * 2 + lax.axis_index("c")
        base_row = wid * BATCH_PER_WORKER * SEQ
        base_batch = wid * BATCH_PER_WORKER

        def writeback_descs(buf, c):
            descs = []
            for b in range(BPC):
                bid = base_batch + c * BPC + b
                descs.append(
                    pltpu.make_async_copy(
                        rows_v.at[buf, pl.ds(b * SEQ, SEQ)],
                        out_hbm.at[bid, pl.ds(0, SEQ), pl.ds(0, EMBED_DIM)],
                        sem_w,
                    )
                )
            return descs

        pltpu.sync_copy(idx_hbm.at[pl.ds(base_row, CHUNK)], idx_v.at[0])

        @pl.loop(0, NUM_CHUNKS // 2)
        def _(g):
            for parity in (0, 1):
                c = g * 2 + parity
                buf = parity
                nxt = 1 - parity
                if parity == 0:
                    pltpu.async_copy(
                        idx_hbm.at[pl.ds(base_row + (c + 1) * CHUNK, CHUNK)],
                        idx_v.at[nxt],
                        sem_i,
                    )
                else:
                    @pl.when(g < NUM_CHUNKS // 2 - 1)
                    def _():
                        pltpu.async_copy(
                            idx_hbm.at[pl.ds(base_row + (c + 1) * CHUNK, CHUNK)],
                            idx_v.at[nxt],
                            sem_i,
                        )

                @pl.when(g >= 1)
                def _():
                    for d in writeback_descs(buf, c - 2):
                        d.wait()

                pltpu.async_copy(
                    table_hbm.at[idx_v.at[buf]], rows_v.at[buf], sem_g
                ).wait()

                for d in writeback_descs(buf, c):
                    d.start()

                if parity == 0:
                    pltpu.make_async_copy(
                        idx_hbm.at[pl.ds(0, CHUNK)], idx_v.at[nxt], sem_i
                    ).wait()
                else:
                    @pl.when(g < NUM_CHUNKS // 2 - 1)
                    def _():
                        pltpu.make_async_copy(
                            idx_hbm.at[pl.ds(0, CHUNK)], idx_v.at[nxt], sem_i
                        ).wait()

        for c in (NUM_CHUNKS - 2, NUM_CHUNKS - 1):
            for d in writeback_descs(c % 2, c):
                d.wait()

    return sc_gather(table, idx)


def kernel(x, table):
    idx = x.reshape(NUM_INDICES)

    @jax.jit
    def run(table, idx):
        mid = _sc_gather(table, idx)
        return mid[:, :SEQ, :EMBED_DIM]

    return run(table, idx)
